# baseline (speedup 1.0000x reference)
import jax
import jax.numpy as jnp
from jax.experimental import pallas as pl
from jax.experimental.pallas import tpu as pltpu

_N = 100000
_K = 384
_M = 128
_B = 2000
_T = _N // _B


def _bn_linear_kernel(x_ref, w_ref, b_ref, g_ref, beta_ref, out_ref,
                      y_ref, sum_ref, sq_ref):
    i = pl.program_id(0)

    @pl.when(i < _T)
    def _compute_phase():
        y = jax.lax.dot_general(
            x_ref[...].astype(jnp.bfloat16), w_ref[...],
            dimension_numbers=(((1,), (1,)), ((), ())),
            preferred_element_type=jnp.float32,
        ) + b_ref[...]
        y_ref[pl.ds(i * _B, _B), :] = y
        ps = jnp.sum(y, axis=0, keepdims=True)
        pq = jnp.sum(y * y, axis=0, keepdims=True)

        @pl.when(i == 0)
        def _():
            sum_ref[...] = ps
            sq_ref[...] = pq

        @pl.when(i > 0)
        def _():
            sum_ref[...] += ps
            sq_ref[...] += pq

    @pl.when(i >= _T)
    def _normalize_phase():
        j = i - _T
        inv_n = 1.0 / _N
        mean = sum_ref[...] * inv_n
        var = sq_ref[...] * inv_n - mean * mean
        scale = g_ref[...] * jax.lax.rsqrt(var + 1e-5)
        shift = beta_ref[...] - mean * scale
        out_ref[...] = y_ref[pl.ds(j * _B, _B), :] * scale + shift


@jax.jit
def kernel(x, W, b, gamma, beta):
    W = W.astype(jnp.bfloat16)
    b2 = b.reshape(1, _M)
    g2 = gamma.reshape(1, _M)
    beta2 = beta.reshape(1, _M)
    return pl.pallas_call(
        _bn_linear_kernel,
        grid=(2 * _T,),
        in_specs=[
            pl.BlockSpec((_B, _K), lambda i: (jnp.minimum(i, _T - 1), 0)),
            pl.BlockSpec((_M, _K), lambda i: (0, 0)),
            pl.BlockSpec((1, _M), lambda i: (0, 0)),
            pl.BlockSpec((1, _M), lambda i: (0, 0)),
            pl.BlockSpec((1, _M), lambda i: (0, 0)),
        ],
        out_specs=pl.BlockSpec((_B, _M), lambda i: (jnp.maximum(i - _T, 0), 0)),
        out_shape=jax.ShapeDtypeStruct((_N, _M), jnp.float32),
        scratch_shapes=[
            pltpu.VMEM((_N, _M), jnp.float32),
            pltpu.VMEM((1, _M), jnp.float32),
            pltpu.VMEM((1, _M), jnp.float32),
        ],
    )(x, W, b2, g2, beta2)


# speedup vs baseline: 1.0345x; 1.0345x over previous
import jax
import jax.numpy as jnp
from jax.experimental import pallas as pl
from jax.experimental.pallas import tpu as pltpu

_N = 100000
_K = 384
_M = 128
_B = 2000
_T = _N // _B


def _bn_linear_kernel(x_ref, w_ref, b_ref, g_ref, beta_ref, out_ref,
                      y_ref, sum_ref, sq_ref):
    i = pl.program_id(0)

    @pl.when(i < _T)
    def _compute_phase():
        y = jax.lax.dot_general(
            x_ref[...], w_ref[...],
            dimension_numbers=(((1,), (1,)), ((), ())),
            preferred_element_type=jnp.float32,
        ) + b_ref[...]
        y_ref[pl.ds(i * _B, _B), :] = y
        ps = jnp.sum(y, axis=0, keepdims=True)
        pq = jnp.sum(y * y, axis=0, keepdims=True)

        @pl.when(i == 0)
        def _():
            sum_ref[...] = ps
            sq_ref[...] = pq

        @pl.when(i > 0)
        def _():
            sum_ref[...] += ps
            sq_ref[...] += pq

    @pl.when(i >= _T)
    def _normalize_phase():
        j = i - _T
        inv_n = 1.0 / _N
        mean = sum_ref[...] * inv_n
        var = sq_ref[...] * inv_n - mean * mean
        scale = g_ref[...] * jax.lax.rsqrt(var + 1e-5)
        shift = beta_ref[...] - mean * scale
        out_ref[...] = y_ref[pl.ds(j * _B, _B), :] * scale + shift


@jax.jit
def kernel(x, W, b, gamma, beta):
    b2 = b.reshape(1, _M)
    g2 = gamma.reshape(1, _M)
    beta2 = beta.reshape(1, _M)
    return pl.pallas_call(
        _bn_linear_kernel,
        grid=(2 * _T,),
        in_specs=[
            pl.BlockSpec((_B, _K), lambda i: (jnp.minimum(i, _T - 1), 0)),
            pl.BlockSpec((_M, _K), lambda i: (0, 0)),
            pl.BlockSpec((1, _M), lambda i: (0, 0)),
            pl.BlockSpec((1, _M), lambda i: (0, 0)),
            pl.BlockSpec((1, _M), lambda i: (0, 0)),
        ],
        out_specs=pl.BlockSpec((_B, _M), lambda i: (jnp.maximum(i - _T, 0), 0)),
        out_shape=jax.ShapeDtypeStruct((_N, _M), jnp.float32),
        scratch_shapes=[
            pltpu.VMEM((_N, _M), jnp.float32),
            pltpu.VMEM((1, _M), jnp.float32),
            pltpu.VMEM((1, _M), jnp.float32),
        ],
    )(x, W, b2, g2, beta2)


# speedup vs baseline: 1.3587x; 1.3135x over previous
import jax
import jax.numpy as jnp
from jax.experimental import pallas as pl
from jax.experimental.pallas import tpu as pltpu

_N = 100000
_K = 384
_M = 128
_B = 4000
_T = _N // _B


def _bn_linear_kernel(x_ref, w_ref, b_ref, g_ref, beta_ref, out_ref,
                      y_ref, sum_ref, sq_ref):
    i = pl.program_id(0)

    @pl.when(i < _T)
    def _compute_phase():
        y = jax.lax.dot_general(
            x_ref[...], w_ref[...],
            dimension_numbers=(((1,), (1,)), ((), ())),
            preferred_element_type=jnp.float32,
        ) + b_ref[...]
        y_ref[pl.ds(i * _B, _B), :] = y.astype(jnp.bfloat16)
        ps = jnp.sum(y, axis=0, keepdims=True)
        pq = jnp.sum(y * y, axis=0, keepdims=True)

        @pl.when(i == 0)
        def _():
            sum_ref[...] = ps
            sq_ref[...] = pq

        @pl.when(i > 0)
        def _():
            sum_ref[...] += ps
            sq_ref[...] += pq

    @pl.when(i >= _T)
    def _normalize_phase():
        j = i - _T
        inv_n = 1.0 / _N
        mean = sum_ref[...] * inv_n
        var = sq_ref[...] * inv_n - mean * mean
        scale = g_ref[...] * jax.lax.rsqrt(var + 1e-5)
        shift = beta_ref[...] - mean * scale
        out_ref[...] = y_ref[pl.ds(j * _B, _B), :].astype(jnp.float32) * scale + shift


@jax.jit
def kernel(x, W, b, gamma, beta):
    b2 = b.reshape(1, _M)
    g2 = gamma.reshape(1, _M)
    beta2 = beta.reshape(1, _M)
    return pl.pallas_call(
        _bn_linear_kernel,
        grid=(2 * _T,),
        in_specs=[
            pl.BlockSpec((_B, _K), lambda i: (jnp.minimum(i, _T - 1), 0)),
            pl.BlockSpec((_M, _K), lambda i: (0, 0)),
            pl.BlockSpec((1, _M), lambda i: (0, 0)),
            pl.BlockSpec((1, _M), lambda i: (0, 0)),
            pl.BlockSpec((1, _M), lambda i: (0, 0)),
        ],
        out_specs=pl.BlockSpec((_B, _M), lambda i: (jnp.maximum(i - _T, 0), 0)),
        out_shape=jax.ShapeDtypeStruct((_N, _M), jnp.float32),
        scratch_shapes=[
            pltpu.VMEM((_N, _M), jnp.bfloat16),
            pltpu.VMEM((1, _M), jnp.float32),
            pltpu.VMEM((1, _M), jnp.float32),
        ],
    )(x, W, b2, g2, beta2)


# speedup vs baseline: 1.4232x; 1.0475x over previous
import jax
import jax.numpy as jnp
from jax.experimental import pallas as pl
from jax.experimental.pallas import tpu as pltpu

_N = 100000
_K = 384
_M = 128
_B = 5000
_T = _N // _B


def _bn_linear_kernel(x_ref, w_ref, b_ref, g_ref, beta_ref, out_ref,
                      y_ref, sum_ref, sq_ref):
    i = pl.program_id(0)

    @pl.when(i < _T)
    def _compute_phase():
        y = jax.lax.dot_general(
            x_ref[...], w_ref[...],
            dimension_numbers=(((1,), (1,)), ((), ())),
            preferred_element_type=jnp.float32,
        ) + b_ref[...]
        y_ref[pl.ds(i * _B, _B), :] = y.astype(jnp.bfloat16)
        ps = jnp.sum(y, axis=0, keepdims=True)
        pq = jnp.sum(y * y, axis=0, keepdims=True)

        @pl.when(i == 0)
        def _():
            sum_ref[...] = ps
            sq_ref[...] = pq

        @pl.when(i > 0)
        def _():
            sum_ref[...] += ps
            sq_ref[...] += pq

    @pl.when(i >= _T)
    def _normalize_phase():
        j = i - _T
        inv_n = 1.0 / _N
        mean = sum_ref[...] * inv_n
        var = sq_ref[...] * inv_n - mean * mean
        scale = g_ref[...] * jax.lax.rsqrt(var + 1e-5)
        shift = beta_ref[...] - mean * scale
        out_ref[...] = y_ref[pl.ds(j * _B, _B), :].astype(jnp.float32) * scale + shift


@jax.jit
def kernel(x, W, b, gamma, beta):
    b2 = b.reshape(1, _M)
    g2 = gamma.reshape(1, _M)
    beta2 = beta.reshape(1, _M)
    return pl.pallas_call(
        _bn_linear_kernel,
        grid=(2 * _T,),
        in_specs=[
            pl.BlockSpec((_B, _K), lambda i: (jnp.minimum(i, _T - 1), 0)),
            pl.BlockSpec((_M, _K), lambda i: (0, 0)),
            pl.BlockSpec((1, _M), lambda i: (0, 0)),
            pl.BlockSpec((1, _M), lambda i: (0, 0)),
            pl.BlockSpec((1, _M), lambda i: (0, 0)),
        ],
        out_specs=pl.BlockSpec((_B, _M), lambda i: (jnp.maximum(i - _T, 0), 0)),
        out_shape=jax.ShapeDtypeStruct((_N, _M), jnp.float32),
        scratch_shapes=[
            pltpu.VMEM((_N, _M), jnp.bfloat16),
            pltpu.VMEM((1, _M), jnp.float32),
            pltpu.VMEM((1, _M), jnp.float32),
        ],
    )(x, W, b2, g2, beta2)
